# Initial kernel scaffold; baseline (speedup 1.0000x reference)
import jax
import jax.numpy as jnp
from jax.experimental import pallas as pl


def kernel(query_emb, memory_embs, WQ, bQ, WK, WV_small, WV_call):
    raise NotImplementedError("write your pallas kernel here")


# speedup vs baseline: 1.3202x; 1.3202x over previous
import jax
import jax.numpy as jnp
from jax.experimental import pallas as pl
from jax.experimental.pallas import tpu as pltpu

D = 768
S = 8192
H = 10
BLOCK_S = 2048


def _b16(x):
    return x.astype(jnp.bfloat16)


def _body(mem_ref, q2d_ref, wq0_ref, wq1_ref, bq0_ref, bq1_ref,
          wk0_ref, wk1_ref, wvt_ref, selt_ref,
          vals_ref, bs_ref, bi_ref,
          q0_s, q1_s, m_s, idx_s, rows_s):
    step = pl.program_id(0)
    nsteps = pl.num_programs(0)

    @pl.when(step == 0)
    def _init():
        q0 = jax.lax.dot_general(
            _b16(q2d_ref[:]), _b16(wq0_ref[:]), (((1,), (1,)), ((), ())),
            preferred_element_type=jnp.float32) + bq0_ref[:]
        q1 = jax.lax.dot_general(
            _b16(q2d_ref[:]), _b16(wq1_ref[:]), (((1,), (1,)), ((), ())),
            preferred_element_type=jnp.float32) + bq1_ref[:]
        q0_s[:] = q0
        q1_s[:] = q1
        m_s[:] = jnp.full((1, H), -jnp.inf, dtype=jnp.float32)
        idx_s[:] = jnp.zeros((1, H), dtype=jnp.int32)
        rows_s[:] = jnp.zeros((D, H), dtype=jnp.float32)

    memb = _b16(mem_ref[:])
    s0 = jax.lax.dot_general(memb, _b16(wk0_ref[:]), (((1,), (1,)), ((), ())),
                             preferred_element_type=jnp.float32)
    s1 = jax.lax.dot_general(memb, _b16(wk1_ref[:]), (((1,), (1,)), ((), ())),
                             preferred_element_type=jnp.float32)
    scores = (_b16(s0).astype(jnp.float32) * _b16(q0_s[:]).astype(jnp.float32)
              + _b16(s1).astype(jnp.float32) * _b16(q1_s[:]).astype(jnp.float32))

    m = jnp.max(scores, axis=0, keepdims=True)
    ii = jax.lax.broadcasted_iota(jnp.int32, scores.shape, 0)
    li = jnp.min(jnp.where(scores == m, ii, BLOCK_S), axis=0, keepdims=True)
    onehot = (ii == li).astype(jnp.bfloat16)
    rows = jax.lax.dot_general(memb, onehot, (((0,), (0,)), ((), ())),
                               preferred_element_type=jnp.float32)

    upd = m > m_s[:]
    m_s[:] = jnp.where(upd, m, m_s[:])
    idx_s[:] = jnp.where(upd, li + step * BLOCK_S, idx_s[:])
    rows_s[:] = jnp.where(upd, rows, rows_s[:])

    @pl.when(step == nsteps - 1)
    def _fin():
        r_sel = jax.lax.dot_general(
            _b16(rows_s[:]), _b16(selt_ref[:]), (((1,), (0,)), ((), ())),
            preferred_element_type=jnp.float32)
        prod = _b16(wvt_ref[:]).astype(jnp.float32) * r_sel
        vals_ref[:] = jnp.sum(prod, axis=0, keepdims=True)
        bs_ref[:] = m_s[:]
        bi_ref[:] = idx_s[:]


def kernel(query_emb, memory_embs, WQ, bQ, WK, WV_small, WV_call):
    q2d = query_emb.reshape(1, D)
    WQ0, WQ1 = WQ[:, 0, :], WQ[:, 1, :]
    bq0, bq1 = bQ[:, 0].reshape(1, H), bQ[:, 1].reshape(1, H)
    WK0, WK1 = WK[:, 0, :], WK[:, 1, :]
    WVt = jnp.concatenate([WV_small[:, 0, :], WV_call], axis=0).T
    SelT = (jnp.arange(H)[:, None] ==
            jnp.minimum(jnp.arange(12)[None, :], 9)).astype(jnp.float32)

    nsteps = S // BLOCK_S
    full = lambda shape: pl.BlockSpec(shape, lambda i: (0, 0))
    vals, bs, bi = pl.pallas_call(
        _body,
        grid=(nsteps,),
        in_specs=[
            pl.BlockSpec((BLOCK_S, D), lambda i: (i, 0)),
            full((1, D)), full((H, D)), full((H, D)),
            full((1, H)), full((1, H)),
            full((H, D)), full((H, D)),
            full((D, 12)), full((H, 12)),
        ],
        out_specs=[full((1, 12)), full((1, H)), full((1, H))],
        out_shape=[
            jax.ShapeDtypeStruct((1, 12), jnp.float32),
            jax.ShapeDtypeStruct((1, H), jnp.float32),
            jax.ShapeDtypeStruct((1, H), jnp.int32),
        ],
        scratch_shapes=[
            pltpu.VMEM((1, H), jnp.float32),
            pltpu.VMEM((1, H), jnp.float32),
            pltpu.VMEM((1, H), jnp.float32),
            pltpu.VMEM((1, H), jnp.int32),
            pltpu.VMEM((D, H), jnp.float32),
        ],
    )(memory_embs, q2d, WQ0, WQ1, bq0, bq1, WK0, WK1, WVt, SelT)
    return vals.reshape(12), bs.reshape(10), bi.reshape(10)
